# baseline (speedup 1.0000x reference)
import functools

import jax
import jax.numpy as jnp
from jax.experimental import pallas as pl
from jax.experimental.pallas import tpu as pltpu

B, N, D = 4096, 512, 16
VEL_W, SUR_W = 0.6, 0.4
GPR = N // 8
LANES = 8 * D
BR = 64
RG = BR * GPR
STEPS = B // BR


def _salience_body(zt_ref, ztm_ref, pt_ref, m_ref, out_ref, idx_ref, acc_ref):
    i = pl.program_id(0)
    zt = zt_ref[...]
    out_ref[...] = zt
    ztm = ztm_ref[...]
    ptile = pt_ref[...]
    mm = m_ref[...]

    diff = zt - ztm
    d2 = diff * diff
    ztp = zt * ptile
    zt2 = zt * zt
    pp2 = ptile * ptile

    dot = functools.partial(
        jnp.dot,
        precision=jax.lax.Precision.HIGHEST,
        preferred_element_type=jnp.float32,
    )
    vel2 = dot(d2, mm)
    zdot = dot(ztp, mm)
    nx2 = dot(zt2, mm)
    ny2 = dot(pp2, mm)

    vel = jnp.sqrt(vel2)
    nx = jnp.sqrt(nx2)
    ny = jnp.sqrt(ny2)
    cos = zdot / jnp.maximum(nx * ny, 1e-8)
    surprise = jnp.clip(1.0 - cos, 0.0, 2.0) / 2.0
    sal = VEL_W * vel + SUR_W * surprise

    sal3 = sal.reshape(BR, GPR, 8)
    mn = jnp.min(sal3, axis=(1, 2), keepdims=True)
    mx = jnp.max(sal3, axis=(1, 2), keepdims=True)
    saln = (sal3 - mn) / (mx - mn + 1e-8)
    bsum = jnp.sum(saln, axis=0)

    @pl.when(i == 0)
    def _init():
        acc_ref[...] = bsum

    @pl.when(i != 0)
    def _accum():
        acc_ref[...] = acc_ref[...] + bsum

    @pl.when(i == STEPS - 1)
    def _finish():
        acc = acc_ref[...]
        m = jnp.max(acc)
        g = jax.lax.broadcasted_iota(jnp.int32, (GPR, 8), 0)
        s = jax.lax.broadcasted_iota(jnp.int32, (GPR, 8), 1)
        eid = g * 8 + s
        idx_ref[0, 0] = jnp.min(jnp.where(acc == m, eid, jnp.int32(2**30)))


def _scatter_body(idx_ref, mt_ref, y_ref, o_ref):
    off = (idx_ref[0] % 8) * D
    lane = jax.lax.broadcasted_iota(jnp.int32, (B, LANES), 1)
    sel = (lane >= off) & (lane < off + D)
    o_ref[...] = jnp.where(sel, mt_ref[...], y_ref[...])


def kernel(z_t, z_tm1, prior, mask_token):
    z2 = z_t.reshape(B * GPR, LANES)
    zm2 = z_tm1.reshape(B * GPR, LANES)
    p2 = prior.reshape(GPR, LANES)
    ptile = jnp.tile(p2, (BR, 1))
    mm = (jax.lax.broadcasted_iota(jnp.int32, (LANES, 8), 0) // D
          == jax.lax.broadcasted_iota(jnp.int32, (LANES, 8), 1)
          ).astype(jnp.float32)

    out_copy, idx = pl.pallas_call(
        _salience_body,
        grid=(STEPS,),
        in_specs=[
            pl.BlockSpec((RG, LANES), lambda i: (i, 0)),
            pl.BlockSpec((RG, LANES), lambda i: (i, 0)),
            pl.BlockSpec((RG, LANES), lambda i: (0, 0)),
            pl.BlockSpec((LANES, 8), lambda i: (0, 0)),
        ],
        out_specs=[
            pl.BlockSpec((RG, LANES), lambda i: (i, 0)),
            pl.BlockSpec(memory_space=pltpu.SMEM),
        ],
        out_shape=[
            jax.ShapeDtypeStruct((B * GPR, LANES), jnp.float32),
            jax.ShapeDtypeStruct((1, 1), jnp.int32),
        ],
        scratch_shapes=[pltpu.VMEM((GPR, 8), jnp.float32)],
    )(z2, zm2, ptile, mm)

    y = out_copy.reshape(B, N * D)
    mt2 = jnp.tile(mask_token.reshape(1, D), (1, 8))
    idx_flat = idx.reshape((1,))

    masked = pl.pallas_call(
        _scatter_body,
        grid_spec=pltpu.PrefetchScalarGridSpec(
            num_scalar_prefetch=1,
            grid=(1,),
            in_specs=[
                pl.BlockSpec((1, LANES), lambda i, sref: (0, 0)),
                pl.BlockSpec((B, LANES), lambda i, sref: (0, sref[0] // 8)),
            ],
            out_specs=pl.BlockSpec((B, LANES), lambda i, sref: (0, sref[0] // 8)),
        ),
        out_shape=jax.ShapeDtypeStruct((B, N * D), jnp.float32),
        input_output_aliases={2: 0},
    )(idx_flat, mt2, y)

    return masked.reshape(B, N, D)
